# baseline (speedup 1.0000x reference)
import functools

import jax
import jax.numpy as jnp
import numpy as np
from jax import lax
from jax.experimental import pallas as pl
from jax.experimental.pallas import tpu as pltpu
from jax.experimental.pallas import tpu_sc as plsc

N = 10000
E = 320000
D_IN = 128
HID = 24
HEADS = 8
EDGE_DIM = 16
OUT = 64
HHID = 4 * HID
HP2 = HHID + 16

NC = 2
NS = 16
EPW = E // (NC * NS)
CH = 128
NFULL = EPW // CH
TAIL = EPW - NFULL * CH
NP = 10240
RPS = NP // NS
ZCH = 128

_SC_PARAMS = pltpu.CompilerParams(use_tc_tiling_on_sc=False,
                                  needs_layout_passes=False)

_R1 = 1000


def _node_body(x_ref, fcw_ref, fcb_ref, wlo_ref, whi_ref, as_ref, ad_ref,
               hplo_ref, hphi_ref, asrc_ref, adst_ref, mxs_ref, mxd_ref):
    i = pl.program_id(0)
    h = jnp.dot(x_ref[...], fcw_ref[...], preferred_element_type=jnp.float32)
    h = h + fcb_ref[...]
    hplo_ref[...] = jnp.dot(h, wlo_ref[...], preferred_element_type=jnp.float32)
    hphi_ref[...] = jnp.dot(h, whi_ref[...], preferred_element_type=jnp.float32)
    a_s = jnp.dot(h, as_ref[...], preferred_element_type=jnp.float32)
    a_d = jnp.dot(h, ad_ref[...], preferred_element_type=jnp.float32)
    asrc_ref[...] = a_s
    adst_ref[...] = a_d
    ms = jnp.max(a_s, axis=0, keepdims=True)
    md = jnp.max(a_d, axis=0, keepdims=True)

    @pl.when(i == 0)
    def _():
        mxs_ref[...] = ms
        mxd_ref[...] = md

    @pl.when(i > 0)
    def _():
        mxs_ref[...] = jnp.maximum(mxs_ref[...], ms)
        mxd_ref[...] = jnp.maximum(mxd_ref[...], md)


def _stage1(x, fc_W, fc_b, Wlo, Whi, As, Ad):
    return pl.pallas_call(
        _node_body,
        grid=(N // _R1,),
        in_specs=[
            pl.BlockSpec((_R1, D_IN), lambda i: (i, 0)),
            pl.BlockSpec((D_IN, HID), lambda i: (0, 0)),
            pl.BlockSpec((1, HID), lambda i: (0, 0)),
            pl.BlockSpec((HID, HP2), lambda i: (0, 0)),
            pl.BlockSpec((HID, HP2), lambda i: (0, 0)),
            pl.BlockSpec((HID, HEADS), lambda i: (0, 0)),
            pl.BlockSpec((HID, HEADS), lambda i: (0, 0)),
        ],
        out_specs=[
            pl.BlockSpec((_R1, HP2), lambda i: (i, 0)),
            pl.BlockSpec((_R1, HP2), lambda i: (i, 0)),
            pl.BlockSpec((_R1, HEADS), lambda i: (i, 0)),
            pl.BlockSpec((_R1, HEADS), lambda i: (i, 0)),
            pl.BlockSpec((1, HEADS), lambda i: (0, 0)),
            pl.BlockSpec((1, HEADS), lambda i: (0, 0)),
        ],
        out_shape=[
            jax.ShapeDtypeStruct((N, HP2), jnp.float32),
            jax.ShapeDtypeStruct((N, HP2), jnp.float32),
            jax.ShapeDtypeStruct((N, HEADS), jnp.float32),
            jax.ShapeDtypeStruct((N, HEADS), jnp.float32),
            jax.ShapeDtypeStruct((1, HEADS), jnp.float32),
            jax.ShapeDtypeStruct((1, HEADS), jnp.float32),
        ],
    )(x, fc_W, fc_b, Wlo, Whi, As, Ad)


_R2 = 8000


def _edge_body(ea_ref, ae_w_ref, ae_ref, mxe_ref):
    i = pl.program_id(0)
    a_e = jnp.dot(ea_ref[...], ae_w_ref[...], preferred_element_type=jnp.float32)
    ae_ref[...] = a_e
    me = jnp.max(a_e, axis=0, keepdims=True)

    @pl.when(i == 0)
    def _():
        mxe_ref[...] = me

    @pl.when(i > 0)
    def _():
        mxe_ref[...] = jnp.maximum(mxe_ref[...], me)


def _stage2(edge_attr, Ae):
    return pl.pallas_call(
        _edge_body,
        grid=(E // _R2,),
        in_specs=[
            pl.BlockSpec((_R2, EDGE_DIM), lambda i: (i, 0)),
            pl.BlockSpec((EDGE_DIM, HEADS), lambda i: (0, 0)),
        ],
        out_specs=[
            pl.BlockSpec((_R2, HEADS), lambda i: (i, 0)),
            pl.BlockSpec((1, HEADS), lambda i: (0, 0)),
        ],
        out_shape=[
            jax.ShapeDtypeStruct((E, HEADS), jnp.float32),
            jax.ShapeDtypeStruct((1, HEADS), jnp.float32),
        ],
    )(edge_attr, Ae)


def _sc_body(src_h, dst_h, aef_h, asrc_h, adst_h, hplo_h, hphi_h, b2_h,
             agglo_o, agghi_o, expv_o,
             agg_s,
             src_iA, dst_iA, src_iB, dst_iB, src_it, dst_it,
             ae_vA, asrc_rA, adst_rA, expv_cA, hp_rA,
             ae_vB, asrc_rB, adst_rB, expv_cB, hp_rB,
             b2_v, gA, gB):
    c = lax.axis_index("c")
    s = lax.axis_index("s")
    w_base = c * (E // NC) + s * EPW

    lane = lax.iota(jnp.int32, 16)
    rowpat = lane // 8
    colpat = lane - rowpat * 8
    blend = lane < 8
    zvec = jnp.zeros((16,), jnp.float32)
    hpatA = [(lane + 16 * j) // HID for j in range(6)]
    hpatB = [4 + (lane + 16 * j) // HID for j in range(6)]

    A = (src_iA, dst_iA, ae_vA, asrc_rA, adst_rA, expv_cA, hp_rA, gA)
    B = (src_iB, dst_iB, ae_vB, asrc_rB, adst_rB, expv_cB, hp_rB, gB)

    pltpu.sync_copy(b2_h, b2_v)
    B2 = b2_v[...]

    def _zrow(e, carry):
        for j in range(7):
            hp_rA[e, pl.ds(j * 16, 16)] = zvec
        return carry

    def _zero_agg():
        lax.fori_loop(0, CH, _zrow, 0)
        for z in range(RPS // ZCH):
            r0 = s * RPS + z * ZCH
            pltpu.sync_copy(hp_rA, agg_s.at[pl.ds(r0, ZCH), :])

    _zero_agg()
    plsc.subcore_barrier()

    def _fills(i, bufs):
        src_i, dst_i, ae_v, asrc_r, adst_r, expv_c, hp_r, g = bufs
        base = w_base + i * CH
        pltpu.sync_copy(src_h.at[pl.ds(base, CH)], src_i)
        pltpu.sync_copy(dst_h.at[pl.ds(base, CH)], dst_i)
        pltpu.sync_copy(aef_h.at[pl.ds(base * HEADS, CH * HEADS)], ae_v)
        pltpu.async_copy(asrc_h.at[src_i], asrc_r, g)
        pltpu.async_copy(adst_h.at[dst_i], adst_r, g)
        pltpu.async_copy(hplo_h.at[src_i], hp_r, g)

    def _mul_rows(expv_c, hp_r, hpat, n, write_den):
        def _ex(e, carry):
            erow = jnp.full((16,), e, jnp.int32)
            for j in range(6):
                av = plsc.load_gather(expv_c, [erow, hpat[j]])
                hp_r[e, pl.ds(j * 16, 16)] = hp_r[e, pl.ds(j * 16, 16)] * av
            if write_den:
                ev = plsc.load_gather(expv_c, [erow, colpat])
                hp_r[e, pl.ds(96, 16)] = jnp.where(blend, ev, zvec)
            return carry

        lax.fori_loop(0, n, _ex, 0)

    def _work(i, bufs):
        src_i, dst_i, ae_v, asrc_r, adst_r, expv_c, hp_r, g = bufs
        base = w_base + i * CH
        pltpu.make_async_copy(asrc_h.at[src_i], asrc_r, g).wait()
        pltpu.make_async_copy(adst_h.at[dst_i], adst_r, g).wait()
        pltpu.make_async_copy(hplo_h.at[src_i], hp_r, g).wait()

        def _ev(j, carry):
            ri = rowpat + 2 * j
            va = plsc.load_gather(asrc_r, [ri, colpat])
            vb = plsc.load_gather(adst_r, [ri, colpat])
            ve = ae_v[pl.ds(j * 16, 16)]
            xs = va + vb + ve
            xs = jnp.maximum(xs, 0.2 * xs)
            xs = jnp.exp(xs - B2)
            plsc.store_scatter(expv_c, [ri, colpat], xs)
            return carry

        lax.fori_loop(0, CH // 2, _ev, 0)
        pltpu.sync_copy(expv_c, expv_o.at[pl.ds(base, CH), :])
        _mul_rows(expv_c, hp_r, hpatA, CH, True)
        pltpu.sync_copy(hp_r, agg_s.at[dst_i], add=True)

    _fills(0, A)

    def _pairA(k, carry):
        c0 = 2 * k
        _fills(c0 + 1, B)
        _work(c0, A)
        _fills(c0 + 2, A)
        _work(c0 + 1, B)
        return carry

    lax.fori_loop(0, NFULL // 2 - 1, _pairA, 0)
    _fills(NFULL - 1, B)
    _work(NFULL - 2, A)
    _work(NFULL - 1, B)

    tbase = w_base + NFULL * CH
    pltpu.sync_copy(src_h.at[pl.ds(tbase, TAIL)], src_it)
    pltpu.sync_copy(dst_h.at[pl.ds(tbase, TAIL)], dst_it)
    pltpu.sync_copy(aef_h.at[pl.ds(tbase * HEADS, TAIL * HEADS)],
                    ae_vA.at[pl.ds(0, TAIL * HEADS)])
    d1 = pltpu.async_copy(asrc_h.at[src_it], asrc_rA.at[pl.ds(0, TAIL), :], gA)
    d2 = pltpu.async_copy(adst_h.at[dst_it], adst_rA.at[pl.ds(0, TAIL), :], gA)
    d3 = pltpu.async_copy(hplo_h.at[src_it], hp_rA.at[pl.ds(0, TAIL), :], gA)
    d1.wait()
    d2.wait()
    d3.wait()

    def _evt(j, carry):
        ri = rowpat + 2 * j
        va = plsc.load_gather(asrc_rA, [ri, colpat])
        vb = plsc.load_gather(adst_rA, [ri, colpat])
        ve = ae_vA[pl.ds(j * 16, 16)]
        xs = va + vb + ve
        xs = jnp.maximum(xs, 0.2 * xs)
        xs = jnp.exp(xs - B2)
        plsc.store_scatter(expv_cA, [ri, colpat], xs)
        return carry

    lax.fori_loop(0, TAIL // 2, _evt, 0)
    pltpu.sync_copy(expv_cA.at[pl.ds(0, TAIL), :], expv_o.at[pl.ds(tbase, TAIL), :])
    _mul_rows(expv_cA, hp_rA, hpatA, TAIL, True)
    pltpu.sync_copy(hp_rA.at[pl.ds(0, TAIL), :], agg_s.at[dst_it], add=True)

    plsc.subcore_barrier()

    for z in range(RPS // ZCH):
        r0 = s * RPS + z * ZCH
        pltpu.sync_copy(agg_s.at[pl.ds(r0, ZCH), :], agglo_o.at[c, pl.ds(r0, ZCH), :])
    _zero_agg()
    plsc.subcore_barrier()

    def _fills2(i, bufs):
        src_i, dst_i, ae_v, asrc_r, adst_r, expv_c, hp_r, g = bufs
        base = w_base + i * CH
        pltpu.sync_copy(src_h.at[pl.ds(base, CH)], src_i)
        pltpu.sync_copy(dst_h.at[pl.ds(base, CH)], dst_i)
        pltpu.sync_copy(expv_o.at[pl.ds(base, CH), :], expv_c)
        pltpu.async_copy(hphi_h.at[src_i], hp_r, g)

    def _work2(i, bufs):
        src_i, dst_i, ae_v, asrc_r, adst_r, expv_c, hp_r, g = bufs
        pltpu.make_async_copy(hphi_h.at[src_i], hp_r, g).wait()
        _mul_rows(expv_c, hp_r, hpatB, CH, False)
        pltpu.sync_copy(hp_r, agg_s.at[dst_i], add=True)

    _fills2(0, A)

    def _pairB(k, carry):
        c0 = 2 * k
        _fills2(c0 + 1, B)
        _work2(c0, A)
        _fills2(c0 + 2, A)
        _work2(c0 + 1, B)
        return carry

    lax.fori_loop(0, NFULL // 2 - 1, _pairB, 0)
    _fills2(NFULL - 1, B)
    _work2(NFULL - 2, A)
    _work2(NFULL - 1, B)

    pltpu.sync_copy(src_h.at[pl.ds(tbase, TAIL)], src_it)
    pltpu.sync_copy(dst_h.at[pl.ds(tbase, TAIL)], dst_it)
    pltpu.sync_copy(expv_o.at[pl.ds(tbase, TAIL), :], expv_cA.at[pl.ds(0, TAIL), :])
    pltpu.async_copy(hphi_h.at[src_it], hp_rA.at[pl.ds(0, TAIL), :], gA).wait()
    _mul_rows(expv_cA, hp_rA, hpatB, TAIL, False)
    pltpu.sync_copy(hp_rA.at[pl.ds(0, TAIL), :], agg_s.at[dst_it], add=True)

    plsc.subcore_barrier()

    for z in range(RPS // ZCH):
        r0 = s * RPS + z * ZCH
        pltpu.sync_copy(agg_s.at[pl.ds(r0, ZCH), :], agghi_o.at[c, pl.ds(r0, ZCH), :])


_sc_call = functools.partial(
    pl.kernel,
    out_type=(
        jax.ShapeDtypeStruct((NC, NP, HP2), jnp.float32),
        jax.ShapeDtypeStruct((NC, NP, HP2), jnp.float32),
        jax.ShapeDtypeStruct((E, HEADS), jnp.float32),
    ),
    mesh=plsc.VectorSubcoreMesh(core_axis_name="c", subcore_axis_name="s",
                                num_cores=NC, num_subcores=NS),
    compiler_params=_SC_PARAMS,
    scratch_types=[
        pltpu.VMEM_SHARED((NP, HP2), jnp.float32),
        pltpu.VMEM((CH,), jnp.int32),
        pltpu.VMEM((CH,), jnp.int32),
        pltpu.VMEM((CH,), jnp.int32),
        pltpu.VMEM((CH,), jnp.int32),
        pltpu.VMEM((TAIL,), jnp.int32),
        pltpu.VMEM((TAIL,), jnp.int32),
        pltpu.VMEM((CH * HEADS,), jnp.float32),
        pltpu.VMEM((CH, HEADS), jnp.float32),
        pltpu.VMEM((CH, HEADS), jnp.float32),
        pltpu.VMEM((CH, HEADS), jnp.float32),
        pltpu.VMEM((CH, HP2), jnp.float32),
        pltpu.VMEM((CH * HEADS,), jnp.float32),
        pltpu.VMEM((CH, HEADS), jnp.float32),
        pltpu.VMEM((CH, HEADS), jnp.float32),
        pltpu.VMEM((CH, HEADS), jnp.float32),
        pltpu.VMEM((CH, HP2), jnp.float32),
        pltpu.VMEM((16,), jnp.float32),
        pltpu.SemaphoreType.DMA,
        pltpu.SemaphoreType.DMA,
    ],
)


def _stage3(src, dst, aef, asrc, adst, hp_lo, hp_hi, b2):
    agglo2, agghi2, _ = _sc_call(_sc_body)(src, dst, aef, asrc, adst,
                                           hp_lo, hp_hi, b2)
    return agglo2, agghi2


_R4 = 1024


def _out_body(agglo_ref, agghi_ref, plo_ref, phi_ref, m_ref,
              ow_ref, ob_ref, cb_ref, y_ref, am_ref):
    i = pl.program_id(0)
    agglo = agglo_ref[0] + agglo_ref[1]
    agghi = agghi_ref[0] + agghi_ref[1]
    den = agglo[:, HHID:HHID + HEADS]
    rec = 1.0 / (den + 1e-16)
    reclo = jnp.dot(rec, plo_ref[...], preferred_element_type=jnp.float32)
    rechi = jnp.dot(rec, phi_ref[...], preferred_element_type=jnp.float32)
    mh = jnp.dot(agglo[:, :HHID] * reclo, m_ref[...],
                 preferred_element_type=jnp.float32)
    mh = mh + jnp.dot(agghi[:, :HHID] * rechi, m_ref[...],
                      preferred_element_type=jnp.float32)
    oc = mh + cb_ref[...]
    oc = jnp.where(oc > 0, oc, jnp.exp(oc) - 1.0)
    y_ref[...] = jnp.dot(oc, ow_ref[...], preferred_element_type=jnp.float32) + ob_ref[...]
    part = jnp.sum(den * rec, axis=0, keepdims=True)

    @pl.when(i == 0)
    def _():
        am_ref[...] = part

    @pl.when(i > 0)
    def _():
        am_ref[...] = am_ref[...] + part

    @pl.when(i == (NP // _R4) - 1)
    def _():
        am_ref[...] = am_ref[...] * (1.0 / E)


def _stage4(agglo2, agghi2, Plo, Phi, M, out_W, out_b, conv_bias):
    return pl.pallas_call(
        _out_body,
        grid=(NP // _R4,),
        in_specs=[
            pl.BlockSpec((NC, _R4, HP2), lambda i: (0, i, 0)),
            pl.BlockSpec((NC, _R4, HP2), lambda i: (0, i, 0)),
            pl.BlockSpec((HEADS, HHID), lambda i: (0, 0)),
            pl.BlockSpec((HEADS, HHID), lambda i: (0, 0)),
            pl.BlockSpec((HHID, HID), lambda i: (0, 0)),
            pl.BlockSpec((HID, OUT), lambda i: (0, 0)),
            pl.BlockSpec((1, OUT), lambda i: (0, 0)),
            pl.BlockSpec((1, HID), lambda i: (0, 0)),
        ],
        out_specs=[
            pl.BlockSpec((_R4, OUT), lambda i: (i, 0)),
            pl.BlockSpec((1, HEADS), lambda i: (0, 0)),
        ],
        out_shape=[
            jax.ShapeDtypeStruct((NP, OUT), jnp.float32),
            jax.ShapeDtypeStruct((1, HEADS), jnp.float32),
        ],
    )(agglo2, agghi2, Plo, Phi, M, out_W, out_b, conv_bias)


def kernel(x, edge_index, edge_attr, fc_W, fc_b, lin_W, att_src, att_dst,
           lin_edge_W, att_edge, conv_bias, out_W, out_b):
    As = jnp.einsum('jhk,hk->jh', lin_W.reshape(HID, HEADS, HID), att_src)
    Ad = jnp.einsum('jhk,hk->jh', lin_W.reshape(HID, HEADS, HID), att_dst)
    Ae = jnp.einsum('dhk,hk->dh', lin_edge_W.reshape(EDGE_DIM, HEADS, HID), att_edge)
    zpad = jnp.zeros((HID, HP2 - HHID), jnp.float32)
    Wlo = jnp.concatenate([lin_W[:, :HHID], zpad], axis=1)
    Whi = jnp.concatenate([lin_W[:, HHID:], zpad], axis=1)

    hp_lo, hp_hi, asrc, adst, mxs, mxd = _stage1(x, fc_W, fc_b.reshape(1, HID),
                                                 Wlo, Whi, As, Ad)
    a_e, mxe = _stage2(edge_attr, Ae)

    b = mxs + mxd + mxe
    b = jnp.maximum(b, 0.2 * b)
    b2 = jnp.concatenate([b, b], axis=1).reshape(16)

    src = edge_index[0]
    dst = edge_index[1]
    agglo2, agghi2 = _stage3(src, dst, a_e.reshape(-1), asrc, adst,
                             hp_lo, hp_hi, b2)

    Plo = np.zeros((HEADS, HHID), np.float32)
    Phi = np.zeros((HEADS, HHID), np.float32)
    for h in range(4):
        Plo[h, h * HID:(h + 1) * HID] = 1.0
        Phi[4 + h, h * HID:(h + 1) * HID] = 1.0
    M = np.zeros((HHID, HID), np.float32)
    for h in range(4):
        M[h * HID:(h + 1) * HID, :] = np.eye(HID, dtype=np.float32) / HEADS
    y, am = _stage4(agglo2, agghi2, jnp.asarray(Plo), jnp.asarray(Phi),
                    jnp.asarray(M), out_W, out_b.reshape(1, OUT),
                    conv_bias.reshape(1, HID))
    return (y[:N], am.reshape(HEADS))


# speedup vs baseline: 1.0451x; 1.0451x over previous
import functools

import jax
import jax.numpy as jnp
import numpy as np
from jax import lax
from jax.experimental import pallas as pl
from jax.experimental.pallas import tpu as pltpu
from jax.experimental.pallas import tpu_sc as plsc

N = 10000
E = 320000
D_IN = 128
HID = 24
HEADS = 8
EDGE_DIM = 16
OUT = 64
HHID = 4 * HID

NC = 2
NS = 16
EPW = E // (NC * NS)
CH = 128
NFULL = EPW // CH
TAIL = EPW - NFULL * CH
NP = 10240
RPS = NP // NS
ZCH = 128

_SC_PARAMS = pltpu.CompilerParams(use_tc_tiling_on_sc=False,
                                  needs_layout_passes=False)

_R1 = 1000


def _node_body(x_ref, fcw_ref, fcb_ref, linw_ref, as_ref, ad_ref,
               hplo_ref, hphi_ref, asrc_ref, adst_ref, mxs_ref, mxd_ref):
    i = pl.program_id(0)
    h = jnp.dot(x_ref[...], fcw_ref[...], preferred_element_type=jnp.float32)
    h = h + fcb_ref[...]
    hp = jnp.dot(h, linw_ref[...], preferred_element_type=jnp.float32)
    hplo_ref[...] = hp[:, :HHID]
    hphi_ref[...] = hp[:, HHID:]
    a_s = jnp.dot(h, as_ref[...], preferred_element_type=jnp.float32)
    a_d = jnp.dot(h, ad_ref[...], preferred_element_type=jnp.float32)
    asrc_ref[...] = a_s
    adst_ref[...] = a_d
    ms = jnp.max(a_s, axis=0, keepdims=True)
    md = jnp.max(a_d, axis=0, keepdims=True)

    @pl.when(i == 0)
    def _():
        mxs_ref[...] = ms
        mxd_ref[...] = md

    @pl.when(i > 0)
    def _():
        mxs_ref[...] = jnp.maximum(mxs_ref[...], ms)
        mxd_ref[...] = jnp.maximum(mxd_ref[...], md)


def _stage1(x, fc_W, fc_b, lin_W, As, Ad):
    return pl.pallas_call(
        _node_body,
        grid=(N // _R1,),
        in_specs=[
            pl.BlockSpec((_R1, D_IN), lambda i: (i, 0)),
            pl.BlockSpec((D_IN, HID), lambda i: (0, 0)),
            pl.BlockSpec((1, HID), lambda i: (0, 0)),
            pl.BlockSpec((HID, HEADS * HID), lambda i: (0, 0)),
            pl.BlockSpec((HID, HEADS), lambda i: (0, 0)),
            pl.BlockSpec((HID, HEADS), lambda i: (0, 0)),
        ],
        out_specs=[
            pl.BlockSpec((_R1, HHID), lambda i: (i, 0)),
            pl.BlockSpec((_R1, HHID), lambda i: (i, 0)),
            pl.BlockSpec((_R1, HEADS), lambda i: (i, 0)),
            pl.BlockSpec((_R1, HEADS), lambda i: (i, 0)),
            pl.BlockSpec((1, HEADS), lambda i: (0, 0)),
            pl.BlockSpec((1, HEADS), lambda i: (0, 0)),
        ],
        out_shape=[
            jax.ShapeDtypeStruct((N, HHID), jnp.float32),
            jax.ShapeDtypeStruct((N, HHID), jnp.float32),
            jax.ShapeDtypeStruct((N, HEADS), jnp.float32),
            jax.ShapeDtypeStruct((N, HEADS), jnp.float32),
            jax.ShapeDtypeStruct((1, HEADS), jnp.float32),
            jax.ShapeDtypeStruct((1, HEADS), jnp.float32),
        ],
    )(x, fc_W, fc_b, lin_W, As, Ad)


_R2 = 8000


def _edge_body(ea_ref, ae_w_ref, ae_ref, mxe_ref):
    i = pl.program_id(0)
    a_e = jnp.dot(ea_ref[...], ae_w_ref[...], preferred_element_type=jnp.float32)
    ae_ref[...] = a_e
    me = jnp.max(a_e, axis=0, keepdims=True)

    @pl.when(i == 0)
    def _():
        mxe_ref[...] = me

    @pl.when(i > 0)
    def _():
        mxe_ref[...] = jnp.maximum(mxe_ref[...], me)


def _stage2(edge_attr, Ae):
    return pl.pallas_call(
        _edge_body,
        grid=(E // _R2,),
        in_specs=[
            pl.BlockSpec((_R2, EDGE_DIM), lambda i: (i, 0)),
            pl.BlockSpec((EDGE_DIM, HEADS), lambda i: (0, 0)),
        ],
        out_specs=[
            pl.BlockSpec((_R2, HEADS), lambda i: (i, 0)),
            pl.BlockSpec((1, HEADS), lambda i: (0, 0)),
        ],
        out_shape=[
            jax.ShapeDtypeStruct((E, HEADS), jnp.float32),
            jax.ShapeDtypeStruct((1, HEADS), jnp.float32),
        ],
    )(edge_attr, Ae)


def _sc1_body(src_h, dst_h, aef_h, asrc_h, adst_h, hp_h, b2_h,
              den_o, agg_o, expv_o,
              den_s, agg_s,
              src_iA, dst_iA, src_iB, dst_iB, src_it, dst_it,
              ae_vA, asrc_rA, adst_rA, expv_cA, hp_rA,
              ae_vB, asrc_rB, adst_rB, expv_cB, hp_rB,
              b2_v, gA, gB):
    c = lax.axis_index("c")
    s = lax.axis_index("s")
    w_base = c * (E // NC) + s * EPW

    lane = lax.iota(jnp.int32, 16)
    rowpat = lane // 8
    colpat = lane - rowpat * 8
    zvec = jnp.zeros((16,), jnp.float32)
    hpat = [(lane + 16 * j) // HID for j in range(6)]

    A = (src_iA, dst_iA, ae_vA, asrc_rA, adst_rA, expv_cA, hp_rA, gA)
    B = (src_iB, dst_iB, ae_vB, asrc_rB, adst_rB, expv_cB, hp_rB, gB)

    pltpu.sync_copy(b2_h, b2_v)
    B2 = b2_v[...]

    def _zrow(e, carry):
        for j in range(6):
            hp_rA[e, pl.ds(j * 16, 16)] = zvec
        return carry

    lax.fori_loop(0, CH, _zrow, 0)

    def _zrow2(j, carry):
        plsc.store_scatter(expv_cA, [rowpat + 2 * j, colpat], zvec)
        return carry

    lax.fori_loop(0, CH // 2, _zrow2, 0)

    for z in range(RPS // ZCH):
        r0 = s * RPS + z * ZCH
        pltpu.sync_copy(hp_rA, agg_s.at[pl.ds(r0, ZCH), :])
        pltpu.sync_copy(expv_cA, den_s.at[pl.ds(r0, ZCH), :])

    plsc.subcore_barrier()

    def _fills(i, bufs):
        src_i, dst_i, ae_v, asrc_r, adst_r, expv_c, hp_r, g = bufs
        base = w_base + i * CH
        pltpu.sync_copy(src_h.at[pl.ds(base, CH)], src_i)
        pltpu.sync_copy(dst_h.at[pl.ds(base, CH)], dst_i)
        pltpu.sync_copy(aef_h.at[pl.ds(base * HEADS, CH * HEADS)], ae_v)
        pltpu.async_copy(asrc_h.at[src_i], asrc_r, g)
        pltpu.async_copy(adst_h.at[dst_i], adst_r, g)
        pltpu.async_copy(hp_h.at[src_i], hp_r, g)

    def _work(i, bufs):
        src_i, dst_i, ae_v, asrc_r, adst_r, expv_c, hp_r, g = bufs
        base = w_base + i * CH
        pltpu.make_async_copy(asrc_h.at[src_i], asrc_r, g).wait()
        pltpu.make_async_copy(adst_h.at[dst_i], adst_r, g).wait()
        pltpu.make_async_copy(hp_h.at[src_i], hp_r, g).wait()

        def _ev(j, carry):
            ri = rowpat + 2 * j
            va = plsc.load_gather(asrc_r, [ri, colpat])
            vb = plsc.load_gather(adst_r, [ri, colpat])
            ve = ae_v[pl.ds(j * 16, 16)]
            xs = va + vb + ve
            xs = jnp.maximum(xs, 0.2 * xs)
            xs = jnp.exp(xs - B2)
            plsc.store_scatter(expv_c, [ri, colpat], xs)
            return carry

        lax.fori_loop(0, CH // 2, _ev, 0)
        pltpu.sync_copy(expv_c, den_s.at[dst_i], add=True)
        pltpu.sync_copy(expv_c, expv_o.at[pl.ds(base, CH), :])

        def _ex(e, carry):
            erow = jnp.full((16,), e, jnp.int32)
            for j in range(6):
                av = plsc.load_gather(expv_c, [erow, hpat[j]])
                hp_r[e, pl.ds(j * 16, 16)] = hp_r[e, pl.ds(j * 16, 16)] * av
            return carry

        lax.fori_loop(0, CH, _ex, 0)
        pltpu.sync_copy(hp_r, agg_s.at[dst_i], add=True)

    _fills(0, A)

    def _pairA(k, carry):
        c0 = 2 * k
        _fills(c0 + 1, B)
        _work(c0, A)
        _fills(c0 + 2, A)
        _work(c0 + 1, B)
        return carry

    lax.fori_loop(0, NFULL // 2 - 1, _pairA, 0)
    _fills(NFULL - 1, B)
    _work(NFULL - 2, A)
    _work(NFULL - 1, B)

    tbase = w_base + NFULL * CH
    pltpu.sync_copy(src_h.at[pl.ds(tbase, TAIL)], src_it)
    pltpu.sync_copy(dst_h.at[pl.ds(tbase, TAIL)], dst_it)
    pltpu.sync_copy(aef_h.at[pl.ds(tbase * HEADS, TAIL * HEADS)],
                    ae_vA.at[pl.ds(0, TAIL * HEADS)])
    d1 = pltpu.async_copy(asrc_h.at[src_it], asrc_rA.at[pl.ds(0, TAIL), :], gA)
    d2 = pltpu.async_copy(adst_h.at[dst_it], adst_rA.at[pl.ds(0, TAIL), :], gA)
    d3 = pltpu.async_copy(hp_h.at[src_it], hp_rA.at[pl.ds(0, TAIL), :], gA)
    d1.wait()
    d2.wait()
    d3.wait()

    def _evt(j, carry):
        ri = rowpat + 2 * j
        va = plsc.load_gather(asrc_rA, [ri, colpat])
        vb = plsc.load_gather(adst_rA, [ri, colpat])
        ve = ae_vA[pl.ds(j * 16, 16)]
        xs = va + vb + ve
        xs = jnp.maximum(xs, 0.2 * xs)
        xs = jnp.exp(xs - B2)
        plsc.store_scatter(expv_cA, [ri, colpat], xs)
        return carry

    lax.fori_loop(0, TAIL // 2, _evt, 0)
    pltpu.sync_copy(expv_cA.at[pl.ds(0, TAIL), :], den_s.at[dst_it], add=True)
    pltpu.sync_copy(expv_cA.at[pl.ds(0, TAIL), :], expv_o.at[pl.ds(tbase, TAIL), :])

    def _ext(e, carry):
        erow = jnp.full((16,), e, jnp.int32)
        for j in range(6):
            av = plsc.load_gather(expv_cA, [erow, hpat[j]])
            hp_rA[e, pl.ds(j * 16, 16)] = hp_rA[e, pl.ds(j * 16, 16)] * av
        return carry

    lax.fori_loop(0, TAIL, _ext, 0)
    pltpu.sync_copy(hp_rA.at[pl.ds(0, TAIL), :], agg_s.at[dst_it], add=True)

    plsc.subcore_barrier()

    for z in range(RPS // ZCH):
        r0 = s * RPS + z * ZCH
        pltpu.sync_copy(agg_s.at[pl.ds(r0, ZCH), :], agg_o.at[c, pl.ds(r0, ZCH), :])
        pltpu.sync_copy(den_s.at[pl.ds(r0, ZCH), :], den_o.at[c, pl.ds(r0, ZCH), :])


_sc1_call = functools.partial(
    pl.kernel,
    out_type=(
        jax.ShapeDtypeStruct((NC, NP, HEADS), jnp.float32),
        jax.ShapeDtypeStruct((NC, NP, HHID), jnp.float32),
        jax.ShapeDtypeStruct((E, HEADS), jnp.float32),
    ),
    mesh=plsc.VectorSubcoreMesh(core_axis_name="c", subcore_axis_name="s",
                                num_cores=NC, num_subcores=NS),
    compiler_params=_SC_PARAMS,
    scratch_types=[
        pltpu.VMEM_SHARED((NP, HEADS), jnp.float32),
        pltpu.VMEM_SHARED((NP, HHID), jnp.float32),
        pltpu.VMEM((CH,), jnp.int32),
        pltpu.VMEM((CH,), jnp.int32),
        pltpu.VMEM((CH,), jnp.int32),
        pltpu.VMEM((CH,), jnp.int32),
        pltpu.VMEM((TAIL,), jnp.int32),
        pltpu.VMEM((TAIL,), jnp.int32),
        pltpu.VMEM((CH * HEADS,), jnp.float32),
        pltpu.VMEM((CH, HEADS), jnp.float32),
        pltpu.VMEM((CH, HEADS), jnp.float32),
        pltpu.VMEM((CH, HEADS), jnp.float32),
        pltpu.VMEM((CH, HHID), jnp.float32),
        pltpu.VMEM((CH * HEADS,), jnp.float32),
        pltpu.VMEM((CH, HEADS), jnp.float32),
        pltpu.VMEM((CH, HEADS), jnp.float32),
        pltpu.VMEM((CH, HEADS), jnp.float32),
        pltpu.VMEM((CH, HHID), jnp.float32),
        pltpu.VMEM((16,), jnp.float32),
        pltpu.SemaphoreType.DMA,
        pltpu.SemaphoreType.DMA,
    ],
)


def _sc2_body(src_h, dst_h, expv_h, hp_h,
              agg_o,
              agg_s,
              src_iA, dst_iA, src_iB, dst_iB, src_it, dst_it,
              expv_cA, hp_rA, expv_cB, hp_rB, gA, gB):
    c = lax.axis_index("c")
    s = lax.axis_index("s")
    w_base = c * (E // NC) + s * EPW

    lane = lax.iota(jnp.int32, 16)
    zvec = jnp.zeros((16,), jnp.float32)
    hpat = [4 + (lane + 16 * j) // HID for j in range(6)]

    A = (src_iA, dst_iA, expv_cA, hp_rA, gA)
    B = (src_iB, dst_iB, expv_cB, hp_rB, gB)

    def _zrow(e, carry):
        for j in range(6):
            hp_rA[e, pl.ds(j * 16, 16)] = zvec
        return carry

    lax.fori_loop(0, CH, _zrow, 0)

    for z in range(RPS // ZCH):
        r0 = s * RPS + z * ZCH
        pltpu.sync_copy(hp_rA, agg_s.at[pl.ds(r0, ZCH), :])

    plsc.subcore_barrier()

    def _fills(i, bufs):
        src_i, dst_i, expv_c, hp_r, g = bufs
        base = w_base + i * CH
        pltpu.sync_copy(src_h.at[pl.ds(base, CH)], src_i)
        pltpu.sync_copy(dst_h.at[pl.ds(base, CH)], dst_i)
        pltpu.sync_copy(expv_h.at[pl.ds(base, CH), :], expv_c)
        pltpu.async_copy(hp_h.at[src_i], hp_r, g)

    def _work(i, bufs):
        src_i, dst_i, expv_c, hp_r, g = bufs
        pltpu.make_async_copy(hp_h.at[src_i], hp_r, g).wait()

        def _ex(e, carry):
            erow = jnp.full((16,), e, jnp.int32)
            for j in range(6):
                av = plsc.load_gather(expv_c, [erow, hpat[j]])
                hp_r[e, pl.ds(j * 16, 16)] = hp_r[e, pl.ds(j * 16, 16)] * av
            return carry

        lax.fori_loop(0, CH, _ex, 0)
        pltpu.sync_copy(hp_r, agg_s.at[dst_i], add=True)

    _fills(0, A)

    def _pair(k, carry):
        c0 = 2 * k
        _fills(c0 + 1, B)
        _work(c0, A)
        _fills(c0 + 2, A)
        _work(c0 + 1, B)
        return carry

    lax.fori_loop(0, NFULL // 2 - 1, _pair, 0)
    _fills(NFULL - 1, B)
    _work(NFULL - 2, A)
    _work(NFULL - 1, B)

    tbase = w_base + NFULL * CH
    pltpu.sync_copy(src_h.at[pl.ds(tbase, TAIL)], src_it)
    pltpu.sync_copy(dst_h.at[pl.ds(tbase, TAIL)], dst_it)
    pltpu.sync_copy(expv_h.at[pl.ds(tbase, TAIL), :], expv_cA.at[pl.ds(0, TAIL), :])
    pltpu.async_copy(hp_h.at[src_it], hp_rA.at[pl.ds(0, TAIL), :], gA).wait()

    def _ext(e, carry):
        erow = jnp.full((16,), e, jnp.int32)
        for j in range(6):
            av = plsc.load_gather(expv_cA, [erow, hpat[j]])
            hp_rA[e, pl.ds(j * 16, 16)] = hp_rA[e, pl.ds(j * 16, 16)] * av
        return carry

    lax.fori_loop(0, TAIL, _ext, 0)
    pltpu.sync_copy(hp_rA.at[pl.ds(0, TAIL), :], agg_s.at[dst_it], add=True)

    plsc.subcore_barrier()

    for z in range(RPS // ZCH):
        r0 = s * RPS + z * ZCH
        pltpu.sync_copy(agg_s.at[pl.ds(r0, ZCH), :], agg_o.at[c, pl.ds(r0, ZCH), :])


_sc2_call = functools.partial(
    pl.kernel,
    out_type=jax.ShapeDtypeStruct((NC, NP, HHID), jnp.float32),
    mesh=plsc.VectorSubcoreMesh(core_axis_name="c", subcore_axis_name="s",
                                num_cores=NC, num_subcores=NS),
    compiler_params=_SC_PARAMS,
    scratch_types=[
        pltpu.VMEM_SHARED((NP, HHID), jnp.float32),
        pltpu.VMEM((CH,), jnp.int32),
        pltpu.VMEM((CH,), jnp.int32),
        pltpu.VMEM((CH,), jnp.int32),
        pltpu.VMEM((CH,), jnp.int32),
        pltpu.VMEM((TAIL,), jnp.int32),
        pltpu.VMEM((TAIL,), jnp.int32),
        pltpu.VMEM((CH, HEADS), jnp.float32),
        pltpu.VMEM((CH, HHID), jnp.float32),
        pltpu.VMEM((CH, HEADS), jnp.float32),
        pltpu.VMEM((CH, HHID), jnp.float32),
        pltpu.SemaphoreType.DMA,
        pltpu.SemaphoreType.DMA,
    ],
)


def _stage3(src, dst, aef, asrc, adst, hp_lo, hp_hi, b2):
    den2, agglo2, expv = _sc1_call(_sc1_body)(src, dst, aef, asrc, adst, hp_lo, b2)
    agghi2 = _sc2_call(_sc2_body)(src, dst, expv, hp_hi)
    return den2, agglo2, agghi2


_R4 = 1024


def _out_body(den_ref, agglo_ref, agghi_ref, plo_ref, phi_ref, m_ref,
              ow_ref, ob_ref, cb_ref, y_ref, am_ref):
    i = pl.program_id(0)
    den = den_ref[0] + den_ref[1]
    agglo = agglo_ref[0] + agglo_ref[1]
    agghi = agghi_ref[0] + agghi_ref[1]
    rec = 1.0 / (den + 1e-16)
    reclo = jnp.dot(rec, plo_ref[...], preferred_element_type=jnp.float32)
    rechi = jnp.dot(rec, phi_ref[...], preferred_element_type=jnp.float32)
    mh = jnp.dot(agglo * reclo, m_ref[...], preferred_element_type=jnp.float32)
    mh = mh + jnp.dot(agghi * rechi, m_ref[...], preferred_element_type=jnp.float32)
    oc = mh + cb_ref[...]
    oc = jnp.where(oc > 0, oc, jnp.exp(oc) - 1.0)
    y_ref[...] = jnp.dot(oc, ow_ref[...], preferred_element_type=jnp.float32) + ob_ref[...]
    part = jnp.sum(den * rec, axis=0, keepdims=True)

    @pl.when(i == 0)
    def _():
        am_ref[...] = part

    @pl.when(i > 0)
    def _():
        am_ref[...] = am_ref[...] + part

    @pl.when(i == (NP // _R4) - 1)
    def _():
        am_ref[...] = am_ref[...] * (1.0 / E)


def _stage4(den2, agglo2, agghi2, Plo, Phi, M, out_W, out_b, conv_bias):
    return pl.pallas_call(
        _out_body,
        grid=(NP // _R4,),
        in_specs=[
            pl.BlockSpec((NC, _R4, HEADS), lambda i: (0, i, 0)),
            pl.BlockSpec((NC, _R4, HHID), lambda i: (0, i, 0)),
            pl.BlockSpec((NC, _R4, HHID), lambda i: (0, i, 0)),
            pl.BlockSpec((HEADS, HHID), lambda i: (0, 0)),
            pl.BlockSpec((HEADS, HHID), lambda i: (0, 0)),
            pl.BlockSpec((HHID, HID), lambda i: (0, 0)),
            pl.BlockSpec((HID, OUT), lambda i: (0, 0)),
            pl.BlockSpec((1, OUT), lambda i: (0, 0)),
            pl.BlockSpec((1, HID), lambda i: (0, 0)),
        ],
        out_specs=[
            pl.BlockSpec((_R4, OUT), lambda i: (i, 0)),
            pl.BlockSpec((1, HEADS), lambda i: (0, 0)),
        ],
        out_shape=[
            jax.ShapeDtypeStruct((NP, OUT), jnp.float32),
            jax.ShapeDtypeStruct((1, HEADS), jnp.float32),
        ],
    )(den2, agglo2, agghi2, Plo, Phi, M, out_W, out_b, conv_bias)


def kernel(x, edge_index, edge_attr, fc_W, fc_b, lin_W, att_src, att_dst,
           lin_edge_W, att_edge, conv_bias, out_W, out_b):
    As = jnp.einsum('jhk,hk->jh', lin_W.reshape(HID, HEADS, HID), att_src)
    Ad = jnp.einsum('jhk,hk->jh', lin_W.reshape(HID, HEADS, HID), att_dst)
    Ae = jnp.einsum('dhk,hk->dh', lin_edge_W.reshape(EDGE_DIM, HEADS, HID), att_edge)

    hp_lo, hp_hi, asrc, adst, mxs, mxd = _stage1(x, fc_W, fc_b.reshape(1, HID),
                                                 lin_W, As, Ad)
    a_e, mxe = _stage2(edge_attr, Ae)

    b = mxs + mxd + mxe
    b = jnp.maximum(b, 0.2 * b)
    b2 = jnp.concatenate([b, b], axis=1).reshape(16)

    src = edge_index[0]
    dst = edge_index[1]
    den2, agglo2, agghi2 = _stage3(src, dst, a_e.reshape(-1), asrc, adst,
                                   hp_lo, hp_hi, b2)

    Plo = np.zeros((HEADS, HHID), np.float32)
    Phi = np.zeros((HEADS, HHID), np.float32)
    for h in range(4):
        Plo[h, h * HID:(h + 1) * HID] = 1.0
        Phi[4 + h, h * HID:(h + 1) * HID] = 1.0
    M = np.zeros((HHID, HID), np.float32)
    for h in range(4):
        M[h * HID:(h + 1) * HID, :] = np.eye(HID, dtype=np.float32) / HEADS
    y, am = _stage4(den2, agglo2, agghi2, jnp.asarray(Plo), jnp.asarray(Phi),
                    jnp.asarray(M), out_W, out_b.reshape(1, OUT),
                    conv_bias.reshape(1, HID))
    return (y[:N], am.reshape(HEADS))


# speedup vs baseline: 1.6666x; 1.5946x over previous
import functools

import jax
import jax.numpy as jnp
import numpy as np
from jax import lax
from jax.experimental import pallas as pl
from jax.experimental.pallas import tpu as pltpu
from jax.experimental.pallas import tpu_sc as plsc

N = 10000
E = 320000
D_IN = 128
HID = 24
HEADS = 8
EDGE_DIM = 16
OUT = 64
HHID = 4 * HID

NC = 2
NS = 16
EPW = E // (NC * NS)
CH = 128
NFULL = EPW // CH
TAIL = EPW - NFULL * CH
NP = 10240
RPS = NP // NS
ZCH = 128

_SC_PARAMS = pltpu.CompilerParams(use_tc_tiling_on_sc=False,
                                  needs_layout_passes=False)

_R1 = 1000


def _node_body(x_ref, fcw_ref, fcb_ref, linw_ref, as_ref, ad_ref,
               hplo_ref, hphi_ref, asrc_ref, adst_ref, mxs_ref, mxd_ref):
    i = pl.program_id(0)
    h = jnp.dot(x_ref[...], fcw_ref[...], preferred_element_type=jnp.float32)
    h = h + fcb_ref[...]
    hp = jnp.dot(h, linw_ref[...], preferred_element_type=jnp.float32)
    hplo_ref[...] = hp[:, :HHID]
    hphi_ref[...] = hp[:, HHID:]
    a_s = jnp.dot(h, as_ref[...], preferred_element_type=jnp.float32)
    a_d = jnp.dot(h, ad_ref[...], preferred_element_type=jnp.float32)
    asrc_ref[...] = a_s
    adst_ref[...] = a_d
    ms = jnp.max(a_s, axis=0, keepdims=True)
    md = jnp.max(a_d, axis=0, keepdims=True)

    @pl.when(i == 0)
    def _():
        mxs_ref[...] = ms
        mxd_ref[...] = md

    @pl.when(i > 0)
    def _():
        mxs_ref[...] = jnp.maximum(mxs_ref[...], ms)
        mxd_ref[...] = jnp.maximum(mxd_ref[...], md)


def _stage1(x, fc_W, fc_b, lin_W, As, Ad):
    return pl.pallas_call(
        _node_body,
        grid=(N // _R1,),
        in_specs=[
            pl.BlockSpec((_R1, D_IN), lambda i: (i, 0)),
            pl.BlockSpec((D_IN, HID), lambda i: (0, 0)),
            pl.BlockSpec((1, HID), lambda i: (0, 0)),
            pl.BlockSpec((HID, HEADS * HID), lambda i: (0, 0)),
            pl.BlockSpec((HID, HEADS), lambda i: (0, 0)),
            pl.BlockSpec((HID, HEADS), lambda i: (0, 0)),
        ],
        out_specs=[
            pl.BlockSpec((_R1, HHID), lambda i: (i, 0)),
            pl.BlockSpec((_R1, HHID), lambda i: (i, 0)),
            pl.BlockSpec((_R1, HEADS), lambda i: (i, 0)),
            pl.BlockSpec((_R1, HEADS), lambda i: (i, 0)),
            pl.BlockSpec((1, HEADS), lambda i: (0, 0)),
            pl.BlockSpec((1, HEADS), lambda i: (0, 0)),
        ],
        out_shape=[
            jax.ShapeDtypeStruct((N, HHID), jnp.float32),
            jax.ShapeDtypeStruct((N, HHID), jnp.float32),
            jax.ShapeDtypeStruct((N, HEADS), jnp.float32),
            jax.ShapeDtypeStruct((N, HEADS), jnp.float32),
            jax.ShapeDtypeStruct((1, HEADS), jnp.float32),
            jax.ShapeDtypeStruct((1, HEADS), jnp.float32),
        ],
    )(x, fc_W, fc_b, lin_W, As, Ad)


_R2 = 8000


def _edge_body(ea_ref, ae_w_ref, ae_ref, mxe_ref):
    i = pl.program_id(0)
    a_e = jnp.dot(ea_ref[...], ae_w_ref[...], preferred_element_type=jnp.float32)
    ae_ref[...] = a_e
    me = jnp.max(a_e, axis=0, keepdims=True)

    @pl.when(i == 0)
    def _():
        mxe_ref[...] = me

    @pl.when(i > 0)
    def _():
        mxe_ref[...] = jnp.maximum(mxe_ref[...], me)


def _stage2(edge_attr, Ae):
    return pl.pallas_call(
        _edge_body,
        grid=(E // _R2,),
        in_specs=[
            pl.BlockSpec((_R2, EDGE_DIM), lambda i: (i, 0)),
            pl.BlockSpec((EDGE_DIM, HEADS), lambda i: (0, 0)),
        ],
        out_specs=[
            pl.BlockSpec((_R2, HEADS), lambda i: (i, 0)),
            pl.BlockSpec((1, HEADS), lambda i: (0, 0)),
        ],
        out_shape=[
            jax.ShapeDtypeStruct((E, HEADS), jnp.float32),
            jax.ShapeDtypeStruct((1, HEADS), jnp.float32),
        ],
    )(edge_attr, Ae)


def _sc1_body(src_h, dst_h, aef_h, asrc_h, adst_h, hp_h, b2_h,
              den_o, agg_o, expv_o,
              den_s, agg_s,
              src_iA, dst_iA, src_iB, dst_iB, src_it, dst_it,
              ae_vA, asrc_rA, adst_rA, expv_cA, hp_rA,
              ae_vB, asrc_rB, adst_rB, expv_cB, hp_rB,
              b2_v, gA, gB):
    c = lax.axis_index("c")
    s = lax.axis_index("s")
    w_base = c * (E // NC) + s * EPW

    lane = lax.iota(jnp.int32, 16)
    rowpat = lane // 8
    colpat = lane - rowpat * 8
    zvec = jnp.zeros((16,), jnp.float32)
    hpat = [(lane + 16 * j) // HID for j in range(6)]

    A = (src_iA, dst_iA, ae_vA, asrc_rA, adst_rA, expv_cA, hp_rA, gA)
    B = (src_iB, dst_iB, ae_vB, asrc_rB, adst_rB, expv_cB, hp_rB, gB)

    pltpu.sync_copy(b2_h, b2_v)
    B2 = b2_v[...]

    def _zrow(e, carry):
        for j in range(6):
            hp_rA[e, pl.ds(j * 16, 16)] = zvec
        return carry

    lax.fori_loop(0, CH, _zrow, 0)

    def _zrow2(j, carry):
        plsc.store_scatter(expv_cA, [rowpat + 2 * j, colpat], zvec)
        return carry

    lax.fori_loop(0, CH // 2, _zrow2, 0)

    for z in range(RPS // ZCH):
        r0 = s * RPS + z * ZCH
        pltpu.sync_copy(hp_rA, agg_s.at[pl.ds(r0, ZCH), :])
        pltpu.sync_copy(expv_cA, den_s.at[pl.ds(r0, ZCH), :])

    plsc.subcore_barrier()

    def _fills(i, bufs):
        src_i, dst_i, ae_v, asrc_r, adst_r, expv_c, hp_r, g = bufs
        base = w_base + i * CH
        pltpu.sync_copy(src_h.at[pl.ds(base, CH)], src_i)
        pltpu.sync_copy(dst_h.at[pl.ds(base, CH)], dst_i)
        pltpu.sync_copy(aef_h.at[pl.ds(base * HEADS, CH * HEADS)], ae_v)
        pltpu.async_copy(asrc_h.at[src_i], asrc_r, g)
        pltpu.async_copy(adst_h.at[dst_i], adst_r, g)
        pltpu.async_copy(hp_h.at[src_i], hp_r, g)

    def _work(i, bufs):
        src_i, dst_i, ae_v, asrc_r, adst_r, expv_c, hp_r, g = bufs
        base = w_base + i * CH
        pltpu.make_async_copy(asrc_h.at[src_i], asrc_r, g).wait()
        pltpu.make_async_copy(adst_h.at[dst_i], adst_r, g).wait()
        pltpu.make_async_copy(hp_h.at[src_i], hp_r, g).wait()

        @plsc.parallel_loop(0, CH // 2, unroll=2)
        def _ev(j):
            ri = rowpat + 2 * j
            va = plsc.load_gather(asrc_r, [ri, colpat])
            vb = plsc.load_gather(adst_r, [ri, colpat])
            ve = ae_v[pl.ds(j * 16, 16)]
            xs = va + vb + ve
            xs = jnp.maximum(xs, 0.2 * xs)
            xs = jnp.exp(xs - B2)
            plsc.store_scatter(expv_c, [ri, colpat], xs)
        pltpu.sync_copy(expv_c, den_s.at[dst_i], add=True)
        pltpu.sync_copy(expv_c, expv_o.at[pl.ds(base, CH), :])

        @plsc.parallel_loop(0, CH, unroll=2)
        def _ex(e):
            erow = jnp.full((16,), e, jnp.int32)
            for j in range(6):
                av = plsc.load_gather(expv_c, [erow, hpat[j]])
                hp_r[e, pl.ds(j * 16, 16)] = hp_r[e, pl.ds(j * 16, 16)] * av

        pltpu.sync_copy(hp_r, agg_s.at[dst_i], add=True)

    _fills(0, A)

    def _pairA(k, carry):
        c0 = 2 * k
        _fills(c0 + 1, B)
        _work(c0, A)
        _fills(c0 + 2, A)
        _work(c0 + 1, B)
        return carry

    lax.fori_loop(0, NFULL // 2 - 1, _pairA, 0)
    _fills(NFULL - 1, B)
    _work(NFULL - 2, A)
    _work(NFULL - 1, B)

    tbase = w_base + NFULL * CH
    pltpu.sync_copy(src_h.at[pl.ds(tbase, TAIL)], src_it)
    pltpu.sync_copy(dst_h.at[pl.ds(tbase, TAIL)], dst_it)
    pltpu.sync_copy(aef_h.at[pl.ds(tbase * HEADS, TAIL * HEADS)],
                    ae_vA.at[pl.ds(0, TAIL * HEADS)])
    d1 = pltpu.async_copy(asrc_h.at[src_it], asrc_rA.at[pl.ds(0, TAIL), :], gA)
    d2 = pltpu.async_copy(adst_h.at[dst_it], adst_rA.at[pl.ds(0, TAIL), :], gA)
    d3 = pltpu.async_copy(hp_h.at[src_it], hp_rA.at[pl.ds(0, TAIL), :], gA)
    d1.wait()
    d2.wait()
    d3.wait()

    def _evt(j, carry):
        ri = rowpat + 2 * j
        va = plsc.load_gather(asrc_rA, [ri, colpat])
        vb = plsc.load_gather(adst_rA, [ri, colpat])
        ve = ae_vA[pl.ds(j * 16, 16)]
        xs = va + vb + ve
        xs = jnp.maximum(xs, 0.2 * xs)
        xs = jnp.exp(xs - B2)
        plsc.store_scatter(expv_cA, [ri, colpat], xs)
        return carry

    lax.fori_loop(0, TAIL // 2, _evt, 0)
    pltpu.sync_copy(expv_cA.at[pl.ds(0, TAIL), :], den_s.at[dst_it], add=True)
    pltpu.sync_copy(expv_cA.at[pl.ds(0, TAIL), :], expv_o.at[pl.ds(tbase, TAIL), :])

    def _ext(e, carry):
        erow = jnp.full((16,), e, jnp.int32)
        for j in range(6):
            av = plsc.load_gather(expv_cA, [erow, hpat[j]])
            hp_rA[e, pl.ds(j * 16, 16)] = hp_rA[e, pl.ds(j * 16, 16)] * av
        return carry

    lax.fori_loop(0, TAIL, _ext, 0)
    pltpu.sync_copy(hp_rA.at[pl.ds(0, TAIL), :], agg_s.at[dst_it], add=True)

    plsc.subcore_barrier()

    for z in range(RPS // ZCH):
        r0 = s * RPS + z * ZCH
        pltpu.sync_copy(agg_s.at[pl.ds(r0, ZCH), :], agg_o.at[c, pl.ds(r0, ZCH), :])
        pltpu.sync_copy(den_s.at[pl.ds(r0, ZCH), :], den_o.at[c, pl.ds(r0, ZCH), :])


_sc1_call = functools.partial(
    pl.kernel,
    out_type=(
        jax.ShapeDtypeStruct((NC, NP, HEADS), jnp.float32),
        jax.ShapeDtypeStruct((NC, NP, HHID), jnp.float32),
        jax.ShapeDtypeStruct((E, HEADS), jnp.float32),
    ),
    mesh=plsc.VectorSubcoreMesh(core_axis_name="c", subcore_axis_name="s",
                                num_cores=NC, num_subcores=NS),
    compiler_params=_SC_PARAMS,
    scratch_types=[
        pltpu.VMEM_SHARED((NP, HEADS), jnp.float32),
        pltpu.VMEM_SHARED((NP, HHID), jnp.float32),
        pltpu.VMEM((CH,), jnp.int32),
        pltpu.VMEM((CH,), jnp.int32),
        pltpu.VMEM((CH,), jnp.int32),
        pltpu.VMEM((CH,), jnp.int32),
        pltpu.VMEM((TAIL,), jnp.int32),
        pltpu.VMEM((TAIL,), jnp.int32),
        pltpu.VMEM((CH * HEADS,), jnp.float32),
        pltpu.VMEM((CH, HEADS), jnp.float32),
        pltpu.VMEM((CH, HEADS), jnp.float32),
        pltpu.VMEM((CH, HEADS), jnp.float32),
        pltpu.VMEM((CH, HHID), jnp.float32),
        pltpu.VMEM((CH * HEADS,), jnp.float32),
        pltpu.VMEM((CH, HEADS), jnp.float32),
        pltpu.VMEM((CH, HEADS), jnp.float32),
        pltpu.VMEM((CH, HEADS), jnp.float32),
        pltpu.VMEM((CH, HHID), jnp.float32),
        pltpu.VMEM((16,), jnp.float32),
        pltpu.SemaphoreType.DMA,
        pltpu.SemaphoreType.DMA,
    ],
)


def _sc2_body(src_h, dst_h, expv_h, hp_h,
              agg_o,
              agg_s,
              src_iA, dst_iA, src_iB, dst_iB, src_it, dst_it,
              expv_cA, hp_rA, expv_cB, hp_rB, gA, gB):
    c = lax.axis_index("c")
    s = lax.axis_index("s")
    w_base = c * (E // NC) + s * EPW

    lane = lax.iota(jnp.int32, 16)
    zvec = jnp.zeros((16,), jnp.float32)
    hpat = [4 + (lane + 16 * j) // HID for j in range(6)]

    A = (src_iA, dst_iA, expv_cA, hp_rA, gA)
    B = (src_iB, dst_iB, expv_cB, hp_rB, gB)

    def _zrow(e, carry):
        for j in range(6):
            hp_rA[e, pl.ds(j * 16, 16)] = zvec
        return carry

    lax.fori_loop(0, CH, _zrow, 0)

    for z in range(RPS // ZCH):
        r0 = s * RPS + z * ZCH
        pltpu.sync_copy(hp_rA, agg_s.at[pl.ds(r0, ZCH), :])

    plsc.subcore_barrier()

    def _fills(i, bufs):
        src_i, dst_i, expv_c, hp_r, g = bufs
        base = w_base + i * CH
        pltpu.sync_copy(src_h.at[pl.ds(base, CH)], src_i)
        pltpu.sync_copy(dst_h.at[pl.ds(base, CH)], dst_i)
        pltpu.sync_copy(expv_h.at[pl.ds(base, CH), :], expv_c)
        pltpu.async_copy(hp_h.at[src_i], hp_r, g)

    def _work(i, bufs):
        src_i, dst_i, expv_c, hp_r, g = bufs
        pltpu.make_async_copy(hp_h.at[src_i], hp_r, g).wait()

        @plsc.parallel_loop(0, CH, unroll=2)
        def _ex(e):
            erow = jnp.full((16,), e, jnp.int32)
            for j in range(6):
                av = plsc.load_gather(expv_c, [erow, hpat[j]])
                hp_r[e, pl.ds(j * 16, 16)] = hp_r[e, pl.ds(j * 16, 16)] * av

        pltpu.sync_copy(hp_r, agg_s.at[dst_i], add=True)

    _fills(0, A)

    def _pair(k, carry):
        c0 = 2 * k
        _fills(c0 + 1, B)
        _work(c0, A)
        _fills(c0 + 2, A)
        _work(c0 + 1, B)
        return carry

    lax.fori_loop(0, NFULL // 2 - 1, _pair, 0)
    _fills(NFULL - 1, B)
    _work(NFULL - 2, A)
    _work(NFULL - 1, B)

    tbase = w_base + NFULL * CH
    pltpu.sync_copy(src_h.at[pl.ds(tbase, TAIL)], src_it)
    pltpu.sync_copy(dst_h.at[pl.ds(tbase, TAIL)], dst_it)
    pltpu.sync_copy(expv_h.at[pl.ds(tbase, TAIL), :], expv_cA.at[pl.ds(0, TAIL), :])
    pltpu.async_copy(hp_h.at[src_it], hp_rA.at[pl.ds(0, TAIL), :], gA).wait()

    def _ext(e, carry):
        erow = jnp.full((16,), e, jnp.int32)
        for j in range(6):
            av = plsc.load_gather(expv_cA, [erow, hpat[j]])
            hp_rA[e, pl.ds(j * 16, 16)] = hp_rA[e, pl.ds(j * 16, 16)] * av
        return carry

    lax.fori_loop(0, TAIL, _ext, 0)
    pltpu.sync_copy(hp_rA.at[pl.ds(0, TAIL), :], agg_s.at[dst_it], add=True)

    plsc.subcore_barrier()

    for z in range(RPS // ZCH):
        r0 = s * RPS + z * ZCH
        pltpu.sync_copy(agg_s.at[pl.ds(r0, ZCH), :], agg_o.at[c, pl.ds(r0, ZCH), :])


_sc2_call = functools.partial(
    pl.kernel,
    out_type=jax.ShapeDtypeStruct((NC, NP, HHID), jnp.float32),
    mesh=plsc.VectorSubcoreMesh(core_axis_name="c", subcore_axis_name="s",
                                num_cores=NC, num_subcores=NS),
    compiler_params=_SC_PARAMS,
    scratch_types=[
        pltpu.VMEM_SHARED((NP, HHID), jnp.float32),
        pltpu.VMEM((CH,), jnp.int32),
        pltpu.VMEM((CH,), jnp.int32),
        pltpu.VMEM((CH,), jnp.int32),
        pltpu.VMEM((CH,), jnp.int32),
        pltpu.VMEM((TAIL,), jnp.int32),
        pltpu.VMEM((TAIL,), jnp.int32),
        pltpu.VMEM((CH, HEADS), jnp.float32),
        pltpu.VMEM((CH, HHID), jnp.float32),
        pltpu.VMEM((CH, HEADS), jnp.float32),
        pltpu.VMEM((CH, HHID), jnp.float32),
        pltpu.SemaphoreType.DMA,
        pltpu.SemaphoreType.DMA,
    ],
)


def _stage3(src, dst, aef, asrc, adst, hp_lo, hp_hi, b2):
    den2, agglo2, expv = _sc1_call(_sc1_body)(src, dst, aef, asrc, adst, hp_lo, b2)
    agghi2 = _sc2_call(_sc2_body)(src, dst, expv, hp_hi)
    return den2, agglo2, agghi2


_R4 = 1024


def _out_body(den_ref, agglo_ref, agghi_ref, plo_ref, phi_ref, m_ref,
              ow_ref, ob_ref, cb_ref, y_ref, am_ref):
    i = pl.program_id(0)
    den = den_ref[0] + den_ref[1]
    agglo = agglo_ref[0] + agglo_ref[1]
    agghi = agghi_ref[0] + agghi_ref[1]
    rec = 1.0 / (den + 1e-16)
    reclo = jnp.dot(rec, plo_ref[...], preferred_element_type=jnp.float32)
    rechi = jnp.dot(rec, phi_ref[...], preferred_element_type=jnp.float32)
    mh = jnp.dot(agglo * reclo, m_ref[...], preferred_element_type=jnp.float32)
    mh = mh + jnp.dot(agghi * rechi, m_ref[...], preferred_element_type=jnp.float32)
    oc = mh + cb_ref[...]
    oc = jnp.where(oc > 0, oc, jnp.exp(oc) - 1.0)
    y_ref[...] = jnp.dot(oc, ow_ref[...], preferred_element_type=jnp.float32) + ob_ref[...]
    part = jnp.sum(den * rec, axis=0, keepdims=True)

    @pl.when(i == 0)
    def _():
        am_ref[...] = part

    @pl.when(i > 0)
    def _():
        am_ref[...] = am_ref[...] + part

    @pl.when(i == (NP // _R4) - 1)
    def _():
        am_ref[...] = am_ref[...] * (1.0 / E)


def _stage4(den2, agglo2, agghi2, Plo, Phi, M, out_W, out_b, conv_bias):
    return pl.pallas_call(
        _out_body,
        grid=(NP // _R4,),
        in_specs=[
            pl.BlockSpec((NC, _R4, HEADS), lambda i: (0, i, 0)),
            pl.BlockSpec((NC, _R4, HHID), lambda i: (0, i, 0)),
            pl.BlockSpec((NC, _R4, HHID), lambda i: (0, i, 0)),
            pl.BlockSpec((HEADS, HHID), lambda i: (0, 0)),
            pl.BlockSpec((HEADS, HHID), lambda i: (0, 0)),
            pl.BlockSpec((HHID, HID), lambda i: (0, 0)),
            pl.BlockSpec((HID, OUT), lambda i: (0, 0)),
            pl.BlockSpec((1, OUT), lambda i: (0, 0)),
            pl.BlockSpec((1, HID), lambda i: (0, 0)),
        ],
        out_specs=[
            pl.BlockSpec((_R4, OUT), lambda i: (i, 0)),
            pl.BlockSpec((1, HEADS), lambda i: (0, 0)),
        ],
        out_shape=[
            jax.ShapeDtypeStruct((NP, OUT), jnp.float32),
            jax.ShapeDtypeStruct((1, HEADS), jnp.float32),
        ],
    )(den2, agglo2, agghi2, Plo, Phi, M, out_W, out_b, conv_bias)


def kernel(x, edge_index, edge_attr, fc_W, fc_b, lin_W, att_src, att_dst,
           lin_edge_W, att_edge, conv_bias, out_W, out_b):
    As = jnp.einsum('jhk,hk->jh', lin_W.reshape(HID, HEADS, HID), att_src)
    Ad = jnp.einsum('jhk,hk->jh', lin_W.reshape(HID, HEADS, HID), att_dst)
    Ae = jnp.einsum('dhk,hk->dh', lin_edge_W.reshape(EDGE_DIM, HEADS, HID), att_edge)

    hp_lo, hp_hi, asrc, adst, mxs, mxd = _stage1(x, fc_W, fc_b.reshape(1, HID),
                                                 lin_W, As, Ad)
    a_e, mxe = _stage2(edge_attr, Ae)

    b = mxs + mxd + mxe
    b = jnp.maximum(b, 0.2 * b)
    b2 = jnp.concatenate([b, b], axis=1).reshape(16)

    src = edge_index[0]
    dst = edge_index[1]
    den2, agglo2, agghi2 = _stage3(src, dst, a_e.reshape(-1), asrc, adst,
                                   hp_lo, hp_hi, b2)

    Plo = np.zeros((HEADS, HHID), np.float32)
    Phi = np.zeros((HEADS, HHID), np.float32)
    for h in range(4):
        Plo[h, h * HID:(h + 1) * HID] = 1.0
        Phi[4 + h, h * HID:(h + 1) * HID] = 1.0
    M = np.zeros((HHID, HID), np.float32)
    for h in range(4):
        M[h * HID:(h + 1) * HID, :] = np.eye(HID, dtype=np.float32) / HEADS
    y, am = _stage4(den2, agglo2, agghi2, jnp.asarray(Plo), jnp.asarray(Phi),
                    jnp.asarray(M), out_W, out_b.reshape(1, OUT),
                    conv_bias.reshape(1, HID))
    return (y[:N], am.reshape(HEADS))


# speedup vs baseline: 1.6671x; 1.0003x over previous
import functools

import jax
import jax.numpy as jnp
import numpy as np
from jax import lax
from jax.experimental import pallas as pl
from jax.experimental.pallas import tpu as pltpu
from jax.experimental.pallas import tpu_sc as plsc

N = 10000
E = 320000
D_IN = 128
HID = 24
HEADS = 8
EDGE_DIM = 16
OUT = 64
HHID = 4 * HID

NC = 2
NS = 16
EPW = E // (NC * NS)
CH = 128
NFULL = EPW // CH
TAIL = EPW - NFULL * CH
NP = 10240
RPS = NP // NS
ZCH = 128

_SC_PARAMS = pltpu.CompilerParams(use_tc_tiling_on_sc=False,
                                  needs_layout_passes=False)

_R1 = 1000


def _node_body(x_ref, fcw_ref, fcb_ref, linw_ref, as_ref, ad_ref,
               hplo_ref, hphi_ref, asrc_ref, adst_ref, mxs_ref, mxd_ref):
    i = pl.program_id(0)
    h = jnp.dot(x_ref[...], fcw_ref[...], preferred_element_type=jnp.float32)
    h = h + fcb_ref[...]
    hp = jnp.dot(h, linw_ref[...], preferred_element_type=jnp.float32)
    hplo_ref[...] = hp[:, :HHID]
    hphi_ref[...] = hp[:, HHID:]
    a_s = jnp.dot(h, as_ref[...], preferred_element_type=jnp.float32)
    a_d = jnp.dot(h, ad_ref[...], preferred_element_type=jnp.float32)
    asrc_ref[...] = a_s
    adst_ref[...] = a_d
    ms = jnp.max(a_s, axis=0, keepdims=True)
    md = jnp.max(a_d, axis=0, keepdims=True)

    @pl.when(i == 0)
    def _():
        mxs_ref[...] = ms
        mxd_ref[...] = md

    @pl.when(i > 0)
    def _():
        mxs_ref[...] = jnp.maximum(mxs_ref[...], ms)
        mxd_ref[...] = jnp.maximum(mxd_ref[...], md)


def _stage1(x, fc_W, fc_b, lin_W, As, Ad):
    return pl.pallas_call(
        _node_body,
        grid=(N // _R1,),
        in_specs=[
            pl.BlockSpec((_R1, D_IN), lambda i: (i, 0)),
            pl.BlockSpec((D_IN, HID), lambda i: (0, 0)),
            pl.BlockSpec((1, HID), lambda i: (0, 0)),
            pl.BlockSpec((HID, HEADS * HID), lambda i: (0, 0)),
            pl.BlockSpec((HID, HEADS), lambda i: (0, 0)),
            pl.BlockSpec((HID, HEADS), lambda i: (0, 0)),
        ],
        out_specs=[
            pl.BlockSpec((_R1, HHID), lambda i: (i, 0)),
            pl.BlockSpec((_R1, HHID), lambda i: (i, 0)),
            pl.BlockSpec((_R1, HEADS), lambda i: (i, 0)),
            pl.BlockSpec((_R1, HEADS), lambda i: (i, 0)),
            pl.BlockSpec((1, HEADS), lambda i: (0, 0)),
            pl.BlockSpec((1, HEADS), lambda i: (0, 0)),
        ],
        out_shape=[
            jax.ShapeDtypeStruct((N, HHID), jnp.float32),
            jax.ShapeDtypeStruct((N, HHID), jnp.float32),
            jax.ShapeDtypeStruct((N, HEADS), jnp.float32),
            jax.ShapeDtypeStruct((N, HEADS), jnp.float32),
            jax.ShapeDtypeStruct((1, HEADS), jnp.float32),
            jax.ShapeDtypeStruct((1, HEADS), jnp.float32),
        ],
    )(x, fc_W, fc_b, lin_W, As, Ad)


_R2 = 8000


def _edge_body(ea_ref, ae_w_ref, ae_ref, mxe_ref):
    i = pl.program_id(0)
    a_e = jnp.dot(ea_ref[...], ae_w_ref[...], preferred_element_type=jnp.float32)
    ae_ref[...] = a_e
    me = jnp.max(a_e, axis=0, keepdims=True)

    @pl.when(i == 0)
    def _():
        mxe_ref[...] = me

    @pl.when(i > 0)
    def _():
        mxe_ref[...] = jnp.maximum(mxe_ref[...], me)


def _stage2(edge_attr, Ae):
    return pl.pallas_call(
        _edge_body,
        grid=(E // _R2,),
        in_specs=[
            pl.BlockSpec((_R2, EDGE_DIM), lambda i: (i, 0)),
            pl.BlockSpec((EDGE_DIM, HEADS), lambda i: (0, 0)),
        ],
        out_specs=[
            pl.BlockSpec((_R2, HEADS), lambda i: (i, 0)),
            pl.BlockSpec((1, HEADS), lambda i: (0, 0)),
        ],
        out_shape=[
            jax.ShapeDtypeStruct((E, HEADS), jnp.float32),
            jax.ShapeDtypeStruct((1, HEADS), jnp.float32),
        ],
    )(edge_attr, Ae)


def _sc1_body(src_h, dst_h, aef_h, asrc_h, adst_h, hp_h, b2_h,
              den_o, agg_o, expv_o,
              den_s, agg_s,
              src_iA, dst_iA, src_iB, dst_iB, src_it, dst_it,
              ae_vA, asrc_rA, adst_rA, expv_cA, hp_rA,
              ae_vB, asrc_rB, adst_rB, expv_cB, hp_rB,
              b2_v, gA, gB):
    c = lax.axis_index("c")
    s = lax.axis_index("s")
    w_base = c * (E // NC) + s * EPW

    lane = lax.iota(jnp.int32, 16)
    rowpat = lane // 8
    colpat = lane - rowpat * 8
    zvec = jnp.zeros((16,), jnp.float32)
    hpat = [(lane + 16 * j) // HID for j in range(6)]

    A = (src_iA, dst_iA, ae_vA, asrc_rA, adst_rA, expv_cA, hp_rA, gA)
    B = (src_iB, dst_iB, ae_vB, asrc_rB, adst_rB, expv_cB, hp_rB, gB)

    pltpu.sync_copy(b2_h, b2_v)
    B2 = b2_v[...]

    def _zrow(e, carry):
        for j in range(6):
            hp_rA[e, pl.ds(j * 16, 16)] = zvec
        return carry

    lax.fori_loop(0, CH, _zrow, 0)

    def _zrow2(j, carry):
        plsc.store_scatter(expv_cA, [rowpat + 2 * j, colpat], zvec)
        return carry

    lax.fori_loop(0, CH // 2, _zrow2, 0)

    for z in range(RPS // ZCH):
        r0 = s * RPS + z * ZCH
        pltpu.sync_copy(hp_rA, agg_s.at[pl.ds(r0, ZCH), :])
        pltpu.sync_copy(expv_cA, den_s.at[pl.ds(r0, ZCH), :])

    plsc.subcore_barrier()

    def _fills(i, bufs):
        src_i, dst_i, ae_v, asrc_r, adst_r, expv_c, hp_r, g = bufs
        base = w_base + i * CH
        pltpu.sync_copy(src_h.at[pl.ds(base, CH)], src_i)
        pltpu.sync_copy(dst_h.at[pl.ds(base, CH)], dst_i)
        pltpu.sync_copy(aef_h.at[pl.ds(base * HEADS, CH * HEADS)], ae_v)
        pltpu.async_copy(asrc_h.at[src_i], asrc_r, g)
        pltpu.async_copy(adst_h.at[dst_i], adst_r, g)
        pltpu.async_copy(hp_h.at[src_i], hp_r, g)

    def _work(i, bufs):
        src_i, dst_i, ae_v, asrc_r, adst_r, expv_c, hp_r, g = bufs
        base = w_base + i * CH
        pltpu.make_async_copy(asrc_h.at[src_i], asrc_r, g).wait()
        pltpu.make_async_copy(adst_h.at[dst_i], adst_r, g).wait()
        pltpu.make_async_copy(hp_h.at[src_i], hp_r, g).wait()

        @plsc.parallel_loop(0, CH // 2, unroll=4)
        def _ev(j):
            ri = rowpat + 2 * j
            va = plsc.load_gather(asrc_r, [ri, colpat])
            vb = plsc.load_gather(adst_r, [ri, colpat])
            ve = ae_v[pl.ds(j * 16, 16)]
            xs = va + vb + ve
            xs = jnp.maximum(xs, 0.2 * xs)
            xs = jnp.exp(xs - B2)
            plsc.store_scatter(expv_c, [ri, colpat], xs)
        pltpu.sync_copy(expv_c, den_s.at[dst_i], add=True)
        pltpu.sync_copy(expv_c, expv_o.at[pl.ds(base, CH), :])

        @plsc.parallel_loop(0, CH, unroll=4)
        def _ex(e):
            erow = jnp.full((16,), e, jnp.int32)
            for j in range(6):
                av = plsc.load_gather(expv_c, [erow, hpat[j]])
                hp_r[e, pl.ds(j * 16, 16)] = hp_r[e, pl.ds(j * 16, 16)] * av

        pltpu.sync_copy(hp_r, agg_s.at[dst_i], add=True)

    _fills(0, A)

    def _pairA(k, carry):
        c0 = 2 * k
        _fills(c0 + 1, B)
        _work(c0, A)
        _fills(c0 + 2, A)
        _work(c0 + 1, B)
        return carry

    lax.fori_loop(0, NFULL // 2 - 1, _pairA, 0)
    _fills(NFULL - 1, B)
    _work(NFULL - 2, A)
    _work(NFULL - 1, B)

    tbase = w_base + NFULL * CH
    pltpu.sync_copy(src_h.at[pl.ds(tbase, TAIL)], src_it)
    pltpu.sync_copy(dst_h.at[pl.ds(tbase, TAIL)], dst_it)
    pltpu.sync_copy(aef_h.at[pl.ds(tbase * HEADS, TAIL * HEADS)],
                    ae_vA.at[pl.ds(0, TAIL * HEADS)])
    d1 = pltpu.async_copy(asrc_h.at[src_it], asrc_rA.at[pl.ds(0, TAIL), :], gA)
    d2 = pltpu.async_copy(adst_h.at[dst_it], adst_rA.at[pl.ds(0, TAIL), :], gA)
    d3 = pltpu.async_copy(hp_h.at[src_it], hp_rA.at[pl.ds(0, TAIL), :], gA)
    d1.wait()
    d2.wait()
    d3.wait()

    def _evt(j, carry):
        ri = rowpat + 2 * j
        va = plsc.load_gather(asrc_rA, [ri, colpat])
        vb = plsc.load_gather(adst_rA, [ri, colpat])
        ve = ae_vA[pl.ds(j * 16, 16)]
        xs = va + vb + ve
        xs = jnp.maximum(xs, 0.2 * xs)
        xs = jnp.exp(xs - B2)
        plsc.store_scatter(expv_cA, [ri, colpat], xs)
        return carry

    lax.fori_loop(0, TAIL // 2, _evt, 0)
    pltpu.sync_copy(expv_cA.at[pl.ds(0, TAIL), :], den_s.at[dst_it], add=True)
    pltpu.sync_copy(expv_cA.at[pl.ds(0, TAIL), :], expv_o.at[pl.ds(tbase, TAIL), :])

    def _ext(e, carry):
        erow = jnp.full((16,), e, jnp.int32)
        for j in range(6):
            av = plsc.load_gather(expv_cA, [erow, hpat[j]])
            hp_rA[e, pl.ds(j * 16, 16)] = hp_rA[e, pl.ds(j * 16, 16)] * av
        return carry

    lax.fori_loop(0, TAIL, _ext, 0)
    pltpu.sync_copy(hp_rA.at[pl.ds(0, TAIL), :], agg_s.at[dst_it], add=True)

    plsc.subcore_barrier()

    for z in range(RPS // ZCH):
        r0 = s * RPS + z * ZCH
        pltpu.sync_copy(agg_s.at[pl.ds(r0, ZCH), :], agg_o.at[c, pl.ds(r0, ZCH), :])
        pltpu.sync_copy(den_s.at[pl.ds(r0, ZCH), :], den_o.at[c, pl.ds(r0, ZCH), :])


_sc1_call = functools.partial(
    pl.kernel,
    out_type=(
        jax.ShapeDtypeStruct((NC, NP, HEADS), jnp.float32),
        jax.ShapeDtypeStruct((NC, NP, HHID), jnp.float32),
        jax.ShapeDtypeStruct((E, HEADS), jnp.float32),
    ),
    mesh=plsc.VectorSubcoreMesh(core_axis_name="c", subcore_axis_name="s",
                                num_cores=NC, num_subcores=NS),
    compiler_params=_SC_PARAMS,
    scratch_types=[
        pltpu.VMEM_SHARED((NP, HEADS), jnp.float32),
        pltpu.VMEM_SHARED((NP, HHID), jnp.float32),
        pltpu.VMEM((CH,), jnp.int32),
        pltpu.VMEM((CH,), jnp.int32),
        pltpu.VMEM((CH,), jnp.int32),
        pltpu.VMEM((CH,), jnp.int32),
        pltpu.VMEM((TAIL,), jnp.int32),
        pltpu.VMEM((TAIL,), jnp.int32),
        pltpu.VMEM((CH * HEADS,), jnp.float32),
        pltpu.VMEM((CH, HEADS), jnp.float32),
        pltpu.VMEM((CH, HEADS), jnp.float32),
        pltpu.VMEM((CH, HEADS), jnp.float32),
        pltpu.VMEM((CH, HHID), jnp.float32),
        pltpu.VMEM((CH * HEADS,), jnp.float32),
        pltpu.VMEM((CH, HEADS), jnp.float32),
        pltpu.VMEM((CH, HEADS), jnp.float32),
        pltpu.VMEM((CH, HEADS), jnp.float32),
        pltpu.VMEM((CH, HHID), jnp.float32),
        pltpu.VMEM((16,), jnp.float32),
        pltpu.SemaphoreType.DMA,
        pltpu.SemaphoreType.DMA,
    ],
)


def _sc2_body(src_h, dst_h, expv_h, hp_h,
              agg_o,
              agg_s,
              src_iA, dst_iA, src_iB, dst_iB, src_it, dst_it,
              expv_cA, hp_rA, expv_cB, hp_rB, gA, gB):
    c = lax.axis_index("c")
    s = lax.axis_index("s")
    w_base = c * (E // NC) + s * EPW

    lane = lax.iota(jnp.int32, 16)
    zvec = jnp.zeros((16,), jnp.float32)
    hpat = [4 + (lane + 16 * j) // HID for j in range(6)]

    A = (src_iA, dst_iA, expv_cA, hp_rA, gA)
    B = (src_iB, dst_iB, expv_cB, hp_rB, gB)

    def _zrow(e, carry):
        for j in range(6):
            hp_rA[e, pl.ds(j * 16, 16)] = zvec
        return carry

    lax.fori_loop(0, CH, _zrow, 0)

    for z in range(RPS // ZCH):
        r0 = s * RPS + z * ZCH
        pltpu.sync_copy(hp_rA, agg_s.at[pl.ds(r0, ZCH), :])

    plsc.subcore_barrier()

    def _fills(i, bufs):
        src_i, dst_i, expv_c, hp_r, g = bufs
        base = w_base + i * CH
        pltpu.sync_copy(src_h.at[pl.ds(base, CH)], src_i)
        pltpu.sync_copy(dst_h.at[pl.ds(base, CH)], dst_i)
        pltpu.sync_copy(expv_h.at[pl.ds(base, CH), :], expv_c)
        pltpu.async_copy(hp_h.at[src_i], hp_r, g)

    def _work(i, bufs):
        src_i, dst_i, expv_c, hp_r, g = bufs
        pltpu.make_async_copy(hp_h.at[src_i], hp_r, g).wait()

        @plsc.parallel_loop(0, CH, unroll=4)
        def _ex(e):
            erow = jnp.full((16,), e, jnp.int32)
            for j in range(6):
                av = plsc.load_gather(expv_c, [erow, hpat[j]])
                hp_r[e, pl.ds(j * 16, 16)] = hp_r[e, pl.ds(j * 16, 16)] * av

        pltpu.sync_copy(hp_r, agg_s.at[dst_i], add=True)

    _fills(0, A)

    def _pair(k, carry):
        c0 = 2 * k
        _fills(c0 + 1, B)
        _work(c0, A)
        _fills(c0 + 2, A)
        _work(c0 + 1, B)
        return carry

    lax.fori_loop(0, NFULL // 2 - 1, _pair, 0)
    _fills(NFULL - 1, B)
    _work(NFULL - 2, A)
    _work(NFULL - 1, B)

    tbase = w_base + NFULL * CH
    pltpu.sync_copy(src_h.at[pl.ds(tbase, TAIL)], src_it)
    pltpu.sync_copy(dst_h.at[pl.ds(tbase, TAIL)], dst_it)
    pltpu.sync_copy(expv_h.at[pl.ds(tbase, TAIL), :], expv_cA.at[pl.ds(0, TAIL), :])
    pltpu.async_copy(hp_h.at[src_it], hp_rA.at[pl.ds(0, TAIL), :], gA).wait()

    def _ext(e, carry):
        erow = jnp.full((16,), e, jnp.int32)
        for j in range(6):
            av = plsc.load_gather(expv_cA, [erow, hpat[j]])
            hp_rA[e, pl.ds(j * 16, 16)] = hp_rA[e, pl.ds(j * 16, 16)] * av
        return carry

    lax.fori_loop(0, TAIL, _ext, 0)
    pltpu.sync_copy(hp_rA.at[pl.ds(0, TAIL), :], agg_s.at[dst_it], add=True)

    plsc.subcore_barrier()

    for z in range(RPS // ZCH):
        r0 = s * RPS + z * ZCH
        pltpu.sync_copy(agg_s.at[pl.ds(r0, ZCH), :], agg_o.at[c, pl.ds(r0, ZCH), :])


_sc2_call = functools.partial(
    pl.kernel,
    out_type=jax.ShapeDtypeStruct((NC, NP, HHID), jnp.float32),
    mesh=plsc.VectorSubcoreMesh(core_axis_name="c", subcore_axis_name="s",
                                num_cores=NC, num_subcores=NS),
    compiler_params=_SC_PARAMS,
    scratch_types=[
        pltpu.VMEM_SHARED((NP, HHID), jnp.float32),
        pltpu.VMEM((CH,), jnp.int32),
        pltpu.VMEM((CH,), jnp.int32),
        pltpu.VMEM((CH,), jnp.int32),
        pltpu.VMEM((CH,), jnp.int32),
        pltpu.VMEM((TAIL,), jnp.int32),
        pltpu.VMEM((TAIL,), jnp.int32),
        pltpu.VMEM((CH, HEADS), jnp.float32),
        pltpu.VMEM((CH, HHID), jnp.float32),
        pltpu.VMEM((CH, HEADS), jnp.float32),
        pltpu.VMEM((CH, HHID), jnp.float32),
        pltpu.SemaphoreType.DMA,
        pltpu.SemaphoreType.DMA,
    ],
)


def _stage3(src, dst, aef, asrc, adst, hp_lo, hp_hi, b2):
    den2, agglo2, expv = _sc1_call(_sc1_body)(src, dst, aef, asrc, adst, hp_lo, b2)
    agghi2 = _sc2_call(_sc2_body)(src, dst, expv, hp_hi)
    return den2, agglo2, agghi2


_R4 = 1024


def _out_body(den_ref, agglo_ref, agghi_ref, plo_ref, phi_ref, m_ref,
              ow_ref, ob_ref, cb_ref, y_ref, am_ref):
    i = pl.program_id(0)
    den = den_ref[0] + den_ref[1]
    agglo = agglo_ref[0] + agglo_ref[1]
    agghi = agghi_ref[0] + agghi_ref[1]
    rec = 1.0 / (den + 1e-16)
    reclo = jnp.dot(rec, plo_ref[...], preferred_element_type=jnp.float32)
    rechi = jnp.dot(rec, phi_ref[...], preferred_element_type=jnp.float32)
    mh = jnp.dot(agglo * reclo, m_ref[...], preferred_element_type=jnp.float32)
    mh = mh + jnp.dot(agghi * rechi, m_ref[...], preferred_element_type=jnp.float32)
    oc = mh + cb_ref[...]
    oc = jnp.where(oc > 0, oc, jnp.exp(oc) - 1.0)
    y_ref[...] = jnp.dot(oc, ow_ref[...], preferred_element_type=jnp.float32) + ob_ref[...]
    part = jnp.sum(den * rec, axis=0, keepdims=True)

    @pl.when(i == 0)
    def _():
        am_ref[...] = part

    @pl.when(i > 0)
    def _():
        am_ref[...] = am_ref[...] + part

    @pl.when(i == (NP // _R4) - 1)
    def _():
        am_ref[...] = am_ref[...] * (1.0 / E)


def _stage4(den2, agglo2, agghi2, Plo, Phi, M, out_W, out_b, conv_bias):
    return pl.pallas_call(
        _out_body,
        grid=(NP // _R4,),
        in_specs=[
            pl.BlockSpec((NC, _R4, HEADS), lambda i: (0, i, 0)),
            pl.BlockSpec((NC, _R4, HHID), lambda i: (0, i, 0)),
            pl.BlockSpec((NC, _R4, HHID), lambda i: (0, i, 0)),
            pl.BlockSpec((HEADS, HHID), lambda i: (0, 0)),
            pl.BlockSpec((HEADS, HHID), lambda i: (0, 0)),
            pl.BlockSpec((HHID, HID), lambda i: (0, 0)),
            pl.BlockSpec((HID, OUT), lambda i: (0, 0)),
            pl.BlockSpec((1, OUT), lambda i: (0, 0)),
            pl.BlockSpec((1, HID), lambda i: (0, 0)),
        ],
        out_specs=[
            pl.BlockSpec((_R4, OUT), lambda i: (i, 0)),
            pl.BlockSpec((1, HEADS), lambda i: (0, 0)),
        ],
        out_shape=[
            jax.ShapeDtypeStruct((NP, OUT), jnp.float32),
            jax.ShapeDtypeStruct((1, HEADS), jnp.float32),
        ],
    )(den2, agglo2, agghi2, Plo, Phi, M, out_W, out_b, conv_bias)


def kernel(x, edge_index, edge_attr, fc_W, fc_b, lin_W, att_src, att_dst,
           lin_edge_W, att_edge, conv_bias, out_W, out_b):
    As = jnp.einsum('jhk,hk->jh', lin_W.reshape(HID, HEADS, HID), att_src)
    Ad = jnp.einsum('jhk,hk->jh', lin_W.reshape(HID, HEADS, HID), att_dst)
    Ae = jnp.einsum('dhk,hk->dh', lin_edge_W.reshape(EDGE_DIM, HEADS, HID), att_edge)

    hp_lo, hp_hi, asrc, adst, mxs, mxd = _stage1(x, fc_W, fc_b.reshape(1, HID),
                                                 lin_W, As, Ad)
    a_e, mxe = _stage2(edge_attr, Ae)

    b = mxs + mxd + mxe
    b = jnp.maximum(b, 0.2 * b)
    b2 = jnp.concatenate([b, b], axis=1).reshape(16)

    src = edge_index[0]
    dst = edge_index[1]
    den2, agglo2, agghi2 = _stage3(src, dst, a_e.reshape(-1), asrc, adst,
                                   hp_lo, hp_hi, b2)

    Plo = np.zeros((HEADS, HHID), np.float32)
    Phi = np.zeros((HEADS, HHID), np.float32)
    for h in range(4):
        Plo[h, h * HID:(h + 1) * HID] = 1.0
        Phi[4 + h, h * HID:(h + 1) * HID] = 1.0
    M = np.zeros((HHID, HID), np.float32)
    for h in range(4):
        M[h * HID:(h + 1) * HID, :] = np.eye(HID, dtype=np.float32) / HEADS
    y, am = _stage4(den2, agglo2, agghi2, jnp.asarray(Plo), jnp.asarray(Phi),
                    jnp.asarray(M), out_W, out_b.reshape(1, OUT),
                    conv_bias.reshape(1, HID))
    return (y[:N], am.reshape(HEADS))
